# Initial kernel scaffold; baseline (speedup 1.0000x reference)
import jax
import jax.numpy as jnp
from jax.experimental import pallas as pl


def kernel(x, edge_index, edge_type, edge_norm, dim, W, root):
    raise NotImplementedError("write your pallas kernel here")


# speedup vs baseline: 33.1396x; 33.1396x over previous
import functools

import jax
import jax.numpy as jnp
from jax import lax
from jax.experimental import pallas as pl
from jax.experimental.pallas import tpu as pltpu
from jax.experimental.pallas import tpu_sc as plsc

_NC = 2
_NS = 16
_NW = _NC * _NS
_LANES = 128


def _sc_coeff_kernel(rows_w, c_size, n_pad):
    c_slice = c_size // _NS

    mesh = plsc.VectorSubcoreMesh(core_axis_name="c", subcore_axis_name="s")

    @functools.partial(
        pl.kernel,
        mesh=mesh,
        out_type=jax.ShapeDtypeStruct((_NC * c_size,), jnp.float32),
        scratch_types=[
            pltpu.VMEM((rows_w, _LANES), jnp.int32),
            pltpu.VMEM((rows_w, _LANES), jnp.int32),
            pltpu.VMEM((rows_w, _LANES), jnp.float32),
            pltpu.VMEM_SHARED((c_size,), jnp.float32),
        ],
    )
    def sc_kernel(src_hbm, typ_hbm, norm_hbm, zeros_hbm, out_hbm,
                  src_v, idx_v, upd_v, acc_sh):
        cid = lax.axis_index("c")
        sid = lax.axis_index("s")
        wid = sid * _NC + cid

        base = wid * rows_w
        pltpu.sync_copy(src_hbm.at[pl.ds(base, rows_w)], src_v)
        pltpu.sync_copy(typ_hbm.at[pl.ds(base, rows_w)], idx_v)
        pltpu.sync_copy(norm_hbm.at[pl.ds(base, rows_w)], upd_v)

        zbase = sid * c_slice
        pltpu.sync_copy(zeros_hbm.at[pl.ds(zbase, c_slice)],
                        acc_sh.at[pl.ds(zbase, c_slice)])

        def _row(i, carry):
            for j in range(_LANES // 16):
                sl = pl.ds(j * 16, 16)
                idx_v[i, sl] = idx_v[i, sl] * n_pad + src_v[i, sl]
            return carry

        lax.fori_loop(0, rows_w, _row, 0)

        plsc.subcore_barrier()

        def _scat(i, carry):
            pltpu.sync_copy(upd_v.at[i], acc_sh.at[idx_v.at[i]], add=True)
            return carry

        lax.fori_loop(0, rows_w, _scat, 0)
        plsc.subcore_barrier()

        pltpu.sync_copy(acc_sh.at[pl.ds(zbase, c_slice)],
                        out_hbm.at[pl.ds(cid * c_size + zbase, c_slice)])

    return sc_kernel


def _tc_body(x_ref, c_ref, w_ref, o_ref):
    x = x_ref[...]
    c = c_ref[0] + c_ref[1]
    nrel = w_ref.shape[0] - 1
    acc = jnp.dot(x, w_ref[nrel], preferred_element_type=jnp.float32)
    for r in range(nrel):
        acc = acc + jnp.dot(x * c[r][:, None], w_ref[r],
                            preferred_element_type=jnp.float32)
    o_ref[...] = acc


def kernel(x, edge_index, edge_type, edge_norm, dim, W, root):
    n, d = x.shape
    r = W.shape[0]
    o = root.shape[1]
    e = edge_type.shape[0]

    blk = 512
    n_pad = -(-n // blk) * blk
    rows_w = -(-e // (_NW * _LANES))
    e_pad = _NW * rows_w * _LANES
    c_size = r * n_pad

    src = edge_index[0]
    pad = e_pad - e
    src_p = jnp.pad(src, (0, pad)).reshape(_NW * rows_w, _LANES)
    typ_p = jnp.pad(edge_type, (0, pad)).reshape(_NW * rows_w, _LANES)
    norm_p = jnp.pad(edge_norm, (0, pad)).reshape(_NW * rows_w, _LANES)
    zeros = jnp.zeros((c_size,), jnp.float32)

    c_parts = _sc_coeff_kernel(rows_w, c_size, n_pad)(src_p, typ_p, norm_p, zeros)
    c_parts = c_parts.reshape(_NC, r, n_pad)

    x_p = jnp.pad(x, ((0, n_pad - n), (0, 0)))
    w_cat = jnp.concatenate([W, root[None]], axis=0)

    grid = n_pad // blk
    out = pl.pallas_call(
        _tc_body,
        grid=(grid,),
        in_specs=[
            pl.BlockSpec((blk, d), lambda i: (i, 0)),
            pl.BlockSpec((_NC, r, blk), lambda i: (0, 0, i)),
            pl.BlockSpec((r + 1, d, o), lambda i: (0, 0, 0)),
        ],
        out_specs=pl.BlockSpec((blk, o), lambda i: (i, 0)),
        out_shape=jax.ShapeDtypeStruct((n_pad, o), jnp.float32),
    )(x_p, c_parts, w_cat)
    return out[:n]
